# Initial kernel scaffold; baseline (speedup 1.0000x reference)
import jax
import jax.numpy as jnp
from jax.experimental import pallas as pl


def kernel(words, tags, word_table, tag_table):
    raise NotImplementedError("write your pallas kernel here")


# speedup vs baseline: 2.8258x; 2.8258x over previous
import functools

import jax
import jax.numpy as jnp
from jax import lax
from jax.experimental import pallas as pl
from jax.experimental.pallas import tpu as pltpu
from jax.experimental.pallas import tpu_sc as plsc

D = 32
NC, NS = 2, 16
NW = NC * NS
CH = 128
S = 4
G = 3


def _emb_body(nch, words_hbm, tags_hbm, wt_hbm, tt_hbm, out_hbm,
              widx, tidx, wrows, trows, sem_g, sem_w):
    wid = lax.axis_index("s") * NC + lax.axis_index("c")
    row0 = wid * (nch * CH)

    pltpu.sync_copy(words_hbm.at[pl.ds(wid * nch, nch)], widx)
    pltpu.sync_copy(tags_hbm.at[pl.ds(wid * nch, nch)], tidx)

    def gather_start(b, j):
        pltpu.async_copy(wt_hbm.at[widx.at[j]], wrows.at[b], sem_g.at[b])
        pltpu.async_copy(tt_hbm.at[tidx.at[j]], trows.at[b], sem_g.at[b])

    def gather_wait(b, j):
        pltpu.make_async_copy(wt_hbm.at[widx.at[j]], wrows.at[b],
                              sem_g.at[b]).wait()
        pltpu.make_async_copy(tt_hbm.at[tidx.at[j]], trows.at[b],
                              sem_g.at[b]).wait()

    def write_start(b, j):
        r0 = row0 + j * CH
        pltpu.async_copy(wrows.at[b], out_hbm.at[pl.ds(r0, CH), 0],
                         sem_w.at[b])
        pltpu.async_copy(trows.at[b], out_hbm.at[pl.ds(r0, CH), 1],
                         sem_w.at[b])

    def write_wait(b, j):
        r0 = row0 + j * CH
        pltpu.make_async_copy(wrows.at[b], out_hbm.at[pl.ds(r0, CH), 0],
                              sem_w.at[b]).wait()
        pltpu.make_async_copy(trows.at[b], out_hbm.at[pl.ds(r0, CH), 1],
                              sem_w.at[b]).wait()

    @pl.loop(0, nch // S)
    def _(g):
        for b in range(S):
            j = g * S + b

            @pl.when(g > 0)
            def _():
                write_wait(b, j - S)

            gather_start(b, j)

            bi = (b - G) % S

            @pl.when(j >= G)
            def _():
                gather_wait(bi, j - G)
                write_start(bi, j - G)

    for t in range(G):
        j = nch - G + t
        bi = j % S
        gather_wait(bi, j)
        write_start(bi, j)
    for b in range(S):
        write_wait(b, nch - S + ((b - (nch - S) % S) % S))


def _build(n):
    assert n % (NW * CH) == 0
    nch = n // (NW * CH)
    mesh = plsc.VectorSubcoreMesh(core_axis_name="c", subcore_axis_name="s")
    return functools.partial(
        pl.kernel,
        out_type=jax.ShapeDtypeStruct((n, 2, D), jnp.float32),
        mesh=mesh,
        compiler_params=pltpu.CompilerParams(use_tc_tiling_on_sc=False),
        scratch_types=[
            pltpu.VMEM((nch, CH), jnp.int32),
            pltpu.VMEM((nch, CH), jnp.int32),
            pltpu.VMEM((S, CH, D), jnp.float32),
            pltpu.VMEM((S, CH, D), jnp.float32),
            pltpu.SemaphoreType.DMA((S,)),
            pltpu.SemaphoreType.DMA((S,)),
        ],
    )(functools.partial(_emb_body, nch))


def kernel(words, tags, word_table, tag_table):
    b, l = words.shape
    n = b * l
    words2d = words.reshape(n // CH, CH)
    tags2d = tags.reshape(n // CH, CH)
    out = _build(n)(words2d, tags2d, word_table, tag_table)
    return out.reshape(b, l, 2 * D)


# speedup vs baseline: 2.8312x; 1.0019x over previous
import functools

import jax
import jax.numpy as jnp
from jax import lax
from jax.experimental import pallas as pl
from jax.experimental.pallas import tpu as pltpu
from jax.experimental.pallas import tpu_sc as plsc

D = 32
NC, NS = 2, 16
NW = NC * NS
CH = 128
S = 8
G = 6


def _emb_body(nch, words_hbm, tags_hbm, wt_hbm, tt_hbm, out_hbm,
              widx, tidx, wrows, trows, sem_g, sem_w):
    wid = lax.axis_index("s") * NC + lax.axis_index("c")
    row0 = wid * (nch * CH)

    pltpu.sync_copy(words_hbm.at[pl.ds(wid * nch, nch)], widx)
    pltpu.sync_copy(tags_hbm.at[pl.ds(wid * nch, nch)], tidx)

    def gather_start(b, j):
        pltpu.async_copy(wt_hbm.at[widx.at[j]], wrows.at[b], sem_g.at[b])
        pltpu.async_copy(tt_hbm.at[tidx.at[j]], trows.at[b], sem_g.at[b])

    def gather_wait(b, j):
        pltpu.make_async_copy(wt_hbm.at[widx.at[j]], wrows.at[b],
                              sem_g.at[b]).wait()
        pltpu.make_async_copy(tt_hbm.at[tidx.at[j]], trows.at[b],
                              sem_g.at[b]).wait()

    def write_start(b, j):
        r0 = row0 + j * CH
        pltpu.async_copy(wrows.at[b], out_hbm.at[pl.ds(r0, CH), 0],
                         sem_w.at[b])
        pltpu.async_copy(trows.at[b], out_hbm.at[pl.ds(r0, CH), 1],
                         sem_w.at[b])

    def write_wait(b, j):
        r0 = row0 + j * CH
        pltpu.make_async_copy(wrows.at[b], out_hbm.at[pl.ds(r0, CH), 0],
                              sem_w.at[b]).wait()
        pltpu.make_async_copy(trows.at[b], out_hbm.at[pl.ds(r0, CH), 1],
                              sem_w.at[b]).wait()

    @pl.loop(0, nch // S)
    def _(g):
        for b in range(S):
            j = g * S + b

            @pl.when(g > 0)
            def _():
                write_wait(b, j - S)

            gather_start(b, j)

            bi = (b - G) % S

            @pl.when(j >= G)
            def _():
                gather_wait(bi, j - G)
                write_start(bi, j - G)

    for t in range(G):
        j = nch - G + t
        bi = j % S
        gather_wait(bi, j)
        write_start(bi, j)
    for b in range(S):
        write_wait(b, nch - S + ((b - (nch - S) % S) % S))


def _build(n):
    assert n % (NW * CH) == 0
    nch = n // (NW * CH)
    mesh = plsc.VectorSubcoreMesh(core_axis_name="c", subcore_axis_name="s")
    return functools.partial(
        pl.kernel,
        out_type=jax.ShapeDtypeStruct((n, 2, D), jnp.float32),
        mesh=mesh,
        compiler_params=pltpu.CompilerParams(use_tc_tiling_on_sc=False),
        scratch_types=[
            pltpu.VMEM((nch, CH), jnp.int32),
            pltpu.VMEM((nch, CH), jnp.int32),
            pltpu.VMEM((S, CH, D), jnp.float32),
            pltpu.VMEM((S, CH, D), jnp.float32),
            pltpu.SemaphoreType.DMA((S,)),
            pltpu.SemaphoreType.DMA((S,)),
        ],
    )(functools.partial(_emb_body, nch))


def kernel(words, tags, word_table, tag_table):
    b, l = words.shape
    n = b * l
    words2d = words.reshape(n // CH, CH)
    tags2d = tags.reshape(n // CH, CH)
    out = _build(n)(words2d, tags2d, word_table, tag_table)
    return out.reshape(b, l, 2 * D)
